# baseline (speedup 1.0000x reference)
import functools

import jax
import jax.numpy as jnp
from jax import lax
from jax.experimental import pallas as pl
from jax.experimental.pallas import tpu as pltpu
from jax.experimental.pallas import tpu_sc as plsc

N = 10000
NPAD = 10240
E = 320000
EP = 100000
D = 128
H = 128
C = 10

NC = 2
NS = 16
NW = NC * NS
ECH = 128
EROWS = E // ECH
RMAX = 80
PH = 48
ERPAD = 2560
RPS = NPAD // NS
EPPAD = 100352
LCH = EPPAD // NW

_f32 = jnp.float32
_i32 = jnp.int32


def _mesh():
  return plsc.VectorSubcoreMesh(core_axis_name="c", subcore_axis_name="s")


def _tile_rows(wid):
  nrows = jnp.where(wid < 24, 80, jnp.where(wid == 31, 76, 72))
  rbase = jnp.where(wid < 24, wid * 80, 1920 + (wid - 24) * 72)
  return nrows, rbase


def _subcore_rows(s):
  nrows = jnp.where(s < 8, 160, jnp.where(s == 15, 156, 152))
  rbase = jnp.where(s < 8, s * 160, 1280 + (s - 8) * 152)
  return nrows, rbase


def _hist_body(dst_hbm, out_hbm, hist_sh, idx_v, val_v, zrow_v, isem):
  c = lax.axis_index("c")
  s = lax.axis_index("s")
  wid = c * NS + s
  nrows, rbase = _tile_rows(wid)

  icp = pltpu.async_copy(dst_hbm.at[pl.ds(rbase, RMAX)], idx_v, isem)

  ones = jnp.ones((16,), _f32)
  zeros = jnp.zeros((16,), _f32)
  for j in range(ECH // 16):
    val_v[pl.ds(j * 16, 16)] = ones

  def zfill(j, _):
    zrow_v[pl.ds(j * 16, 16)] = zeros
    return 0
  lax.fori_loop(0, RPS // 16, zfill, 0)

  pltpu.sync_copy(zrow_v, hist_sh.at[pl.ds(s * RPS, RPS)])
  icp.wait()
  plsc.subcore_barrier()

  def _drain():
    pltpu.make_async_copy(val_v, hist_sh.at[idx_v.at[0]], isem).wait()

  def chunk(i, _):
    pltpu.async_copy(val_v, hist_sh.at[idx_v.at[i]], isem, add=True)

    @pl.when(i >= 8)
    def _():
      _drain()
    return 0
  lax.fori_loop(0, nrows, chunk, 0)

  def tail(_k, _):
    _drain()
    return 0
  lax.fori_loop(0, jnp.minimum(nrows, 8), tail, 0)

  plsc.subcore_barrier()
  pltpu.sync_copy(hist_sh.at[pl.ds(s * RPS, RPS)], zrow_v)
  pltpu.sync_copy(zrow_v, out_hbm.at[c, pl.ds(s * RPS, RPS)])


def _hist_call(dst2):
  k = pl.kernel(
      _hist_body,
      out_type=jax.ShapeDtypeStruct((NC, NPAD), _f32),
      mesh=_mesh(),
      scratch_types=[
          pltpu.VMEM_SHARED((NPAD,), _f32),
          pltpu.VMEM((RMAX, ECH), _i32),
          pltpu.VMEM((ECH,), _f32),
          pltpu.VMEM((RPS,), _f32),
          pltpu.SemaphoreType.DMA,
      ],
  )
  return k(dst2)


def _seg_body(g_hbm, src_hbm, dst_hbm, out_hbm, s_sh, isrc_v, idst_v,
              rows_a, rows_b, isem, ga_sem, gb_sem, sa_sem, sb_sem):
  c = lax.axis_index("c")
  s = lax.axis_index("s")
  wid = c * NS + s
  nrows, rbase = _tile_rows(wid)

  pltpu.async_copy(src_hbm.at[pl.ds(rbase, PH)], isrc_v, isem)
  pltpu.async_copy(dst_hbm.at[pl.ds(rbase, PH)], idst_v, isem)

  zeros = jnp.zeros((16,), _f32)

  def zfill(t, _):
    rows_a[t // 8, pl.ds((t % 8) * 16, 16)] = zeros
    return 0
  lax.fori_loop(0, ECH * 8, zfill, 0)
  for p in range(RPS // ECH):
    pltpu.sync_copy(rows_a, s_sh.at[pl.ds(s * RPS + p * ECH, ECH)])
  pltpu.make_async_copy(src_hbm.at[pl.ds(rbase, PH)], isrc_v, isem).wait()
  pltpu.make_async_copy(dst_hbm.at[pl.ds(rbase, PH)], idst_v, isem).wait()
  plsc.subcore_barrier()

  def _gather(i, buf, sem):
    pltpu.async_copy(g_hbm.at[isrc_v.at[i]], buf, sem)

  def _scatter(i, buf, sem):
    pltpu.async_copy(buf, s_sh.at[idst_v.at[i]], sem, add=True)

  def _wait(buf, sem):
    pltpu.make_async_copy(g_hbm.at[isrc_v.at[0]], buf, sem).wait()

  def _run_phase(npairs):
    _gather(0, rows_a, ga_sem)
    _gather(1, rows_b, gb_sem)

    def pair(j, _):
      i0 = 2 * j
      i1 = i0 + 1
      _wait(rows_a, ga_sem)
      _scatter(i0, rows_a, sa_sem)
      _wait(rows_b, gb_sem)
      _scatter(i1, rows_b, sb_sem)

      @pl.when(j < npairs - 1)
      def _():
        _wait(rows_a, sa_sem)
        _gather(i0 + 2, rows_a, ga_sem)
        _wait(rows_b, sb_sem)
        _gather(i0 + 3, rows_b, gb_sem)

      @pl.when(j == npairs - 1)
      def _():
        _wait(rows_a, sa_sem)
        _wait(rows_b, sb_sem)
      return 0
    lax.fori_loop(0, npairs, pair, 0)

  _run_phase(PH // 2)
  pltpu.sync_copy(src_hbm.at[pl.ds(rbase + PH, PH)], isrc_v)
  pltpu.sync_copy(dst_hbm.at[pl.ds(rbase + PH, PH)], idst_v)
  _run_phase((nrows - PH) // 2)

  plsc.subcore_barrier()
  for p in range(RPS // ECH):
    r0 = s * RPS + p * ECH
    pltpu.sync_copy(s_sh.at[pl.ds(r0, ECH)], rows_a)
    pltpu.sync_copy(rows_a, out_hbm.at[c, pl.ds(r0, ECH)])


def _seg_call(g, src2, dst2):
  k = pl.kernel(
      _seg_body,
      out_type=jax.ShapeDtypeStruct((NC, NPAD, H), _f32),
      mesh=_mesh(),
      scratch_types=[
          pltpu.VMEM_SHARED((NPAD, H), _f32),
          pltpu.VMEM((PH, ECH), _i32),
          pltpu.VMEM((PH, ECH), _i32),
          pltpu.VMEM((ECH, H), _f32),
          pltpu.VMEM((ECH, H), _f32),
          pltpu.SemaphoreType.DMA,
          pltpu.SemaphoreType.DMA,
          pltpu.SemaphoreType.DMA,
          pltpu.SemaphoreType.DMA,
          pltpu.SemaphoreType.DMA,
      ],
  )
  return k(g, src2, dst2)


def _link_body(u_hbm, v_hbm, ps_hbm, pd_hbm, ns_hbm, nd_hbm,
               lp_hbm, ln_hbm, u_v, v_v, is_v, id_v, ob_v):
  c = lax.axis_index("c")
  s = lax.axis_index("s")
  wid = c * NS + s
  base = wid * LCH

  pltpu.sync_copy(u_hbm, u_v)
  pltpu.sync_copy(v_hbm, v_v)

  for (sh, dh, oh) in ((ps_hbm, pd_hbm, lp_hbm), (ns_hbm, nd_hbm, ln_hbm)):
    pltpu.sync_copy(sh.at[pl.ds(base, LCH)], is_v)
    pltpu.sync_copy(dh.at[pl.ds(base, LCH)], id_v)

    def gath(j, _):
      iu = is_v[pl.ds(j * 16, 16)]
      iv = id_v[pl.ds(j * 16, 16)]
      a = plsc.load_gather(u_v, [iu])
      b = plsc.load_gather(v_v, [iv])
      ob_v[pl.ds(j * 16, 16)] = a + b
      return 0
    lax.fori_loop(0, LCH // 16, gath, 0)
    pltpu.sync_copy(ob_v, oh.at[pl.ds(base, LCH)])


def _link_call(u, v, ps, pd, ns_, nd):
  k = pl.kernel(
      _link_body,
      out_type=[jax.ShapeDtypeStruct((EPPAD,), _f32),
                jax.ShapeDtypeStruct((EPPAD,), _f32)],
      mesh=_mesh(),
      scratch_types=[
          pltpu.VMEM((NPAD,), _f32),
          pltpu.VMEM((NPAD,), _f32),
          pltpu.VMEM((LCH,), _i32),
          pltpu.VMEM((LCH,), _i32),
          pltpu.VMEM((LCH,), _f32),
      ],
      compiler_params=pltpu.CompilerParams(needs_layout_passes=False),
  )
  return k(u, v, ps, pd, ns_, nd)


_BLK = 1024
_GRID = NPAD // _BLK


def _dinv_of(hist_ref):
  return lax.rsqrt(1.0 + hist_ref[0, :] + hist_ref[1, :])[:, None]


def _tc1_body(x_ref, w1_ref, hist_ref, g1_ref):
  h = jnp.dot(x_ref[...], w1_ref[...], preferred_element_type=_f32)
  g1_ref[...] = h * _dinv_of(hist_ref)


def _tc1_call(x_pad, w1, hist):
  return pl.pallas_call(
      _tc1_body,
      grid=(_GRID,),
      in_specs=[
          pl.BlockSpec((_BLK, D), lambda i: (i, 0)),
          pl.BlockSpec((D, H), lambda i: (0, 0)),
          pl.BlockSpec((NC, _BLK), lambda i: (0, i)),
      ],
      out_specs=pl.BlockSpec((_BLK, H), lambda i: (i, 0)),
      out_shape=jax.ShapeDtypeStruct((NPAD, H), _f32),
  )(x_pad, w1, hist)


def _tc2_body(s_ref, g1_ref, hist_ref, b1_ref, w2_ref, g2_ref):
  dinv = _dinv_of(hist_ref)
  o1 = jnp.maximum(
      dinv * (s_ref[0] + s_ref[1] + g1_ref[...]) + b1_ref[...], 0.0)
  h2 = jnp.dot(o1, w2_ref[...], preferred_element_type=_f32)
  g2_ref[...] = h2 * dinv


def _tc2_call(s1, g1, hist, b1, w2):
  return pl.pallas_call(
      _tc2_body,
      grid=(_GRID,),
      in_specs=[
          pl.BlockSpec((NC, _BLK, H), lambda i: (0, i, 0)),
          pl.BlockSpec((_BLK, H), lambda i: (i, 0)),
          pl.BlockSpec((NC, _BLK), lambda i: (0, i)),
          pl.BlockSpec((1, H), lambda i: (0, 0)),
          pl.BlockSpec((H, H), lambda i: (0, 0)),
      ],
      out_specs=pl.BlockSpec((_BLK, H), lambda i: (i, 0)),
      out_shape=jax.ShapeDtypeStruct((NPAD, H), _f32),
  )(s1, g1, hist, b1, w2)


def _tc3_body(s_ref, g2_ref, hist_ref, b2_ref, wc_ref, bc_ref,
              wl_ref, bl_ref, np_ref, u_ref, v_ref):
  dinv = _dinv_of(hist_ref)
  h = dinv * (s_ref[0] + s_ref[1] + g2_ref[...]) + b2_ref[...]
  nh = jnp.maximum(h, 0.0)
  logits = jnp.dot(nh, wc_ref[...], preferred_element_type=_f32) + bc_ref[...]
  m = jnp.max(logits, axis=1, keepdims=True)
  lse = jnp.log(jnp.sum(jnp.exp(logits - m), axis=1, keepdims=True)) + m
  np_ref[...] = logits - lse
  uv = jnp.dot(h, wl_ref[...], preferred_element_type=_f32)
  u_ref[...] = uv[:, 0:1] + bl_ref[...]
  v_ref[...] = uv[:, 1:2]


def _tc3_call(s2, g2, hist, b2, wc, bc, wl2, bl):
  return pl.pallas_call(
      _tc3_body,
      grid=(_GRID,),
      in_specs=[
          pl.BlockSpec((NC, _BLK, H), lambda i: (0, i, 0)),
          pl.BlockSpec((_BLK, H), lambda i: (i, 0)),
          pl.BlockSpec((NC, _BLK), lambda i: (0, i)),
          pl.BlockSpec((1, H), lambda i: (0, 0)),
          pl.BlockSpec((H, C), lambda i: (0, 0)),
          pl.BlockSpec((1, C), lambda i: (0, 0)),
          pl.BlockSpec((H, 2), lambda i: (0, 0)),
          pl.BlockSpec((1, 1), lambda i: (0, 0)),
      ],
      out_specs=[
          pl.BlockSpec((_BLK, C), lambda i: (i, 0)),
          pl.BlockSpec((_BLK, 1), lambda i: (i, 0)),
          pl.BlockSpec((_BLK, 1), lambda i: (i, 0)),
      ],
      out_shape=[
          jax.ShapeDtypeStruct((NPAD, C), _f32),
          jax.ShapeDtypeStruct((NPAD, 1), _f32),
          jax.ShapeDtypeStruct((NPAD, 1), _f32),
      ],
  )(s2, g2, hist, b2, wc, bc, wl2, bl)


def kernel(x, edge_index, edge_index_pos, edge_index_neg,
           W1, b1, W2, b2, Wc, bc, Wl, bl):
  src2 = jnp.pad(edge_index[0].reshape(EROWS, ECH),
                 ((0, ERPAD - EROWS), (0, 0)))
  dst2 = jnp.pad(edge_index[1].reshape(EROWS, ECH),
                 ((0, ERPAD - EROWS), (0, 0)))
  x_pad = jnp.pad(x, ((0, NPAD - N), (0, 0)))

  hist = _hist_call(dst2)

  g1 = _tc1_call(x_pad, W1, hist)
  s1 = _seg_call(g1, src2, dst2)
  g2 = _tc2_call(s1, g1, hist, b1.reshape(1, H), W2)
  s2 = _seg_call(g2, src2, dst2)

  wl2 = jnp.concatenate([Wl[:H], Wl[H:]], axis=1)
  node_pad, u2, v2 = _tc3_call(s2, g2, hist, b2.reshape(1, H), Wc,
                               bc.reshape(1, C), wl2, bl.reshape(1, 1))

  pad = (0, EPPAD - EP)
  lp_pad, ln_pad = _link_call(
      u2[:, 0], v2[:, 0],
      jnp.pad(edge_index_pos[0], pad), jnp.pad(edge_index_pos[1], pad),
      jnp.pad(edge_index_neg[0], pad), jnp.pad(edge_index_neg[1], pad))

  return (node_pad[:N], lp_pad[:EP], ln_pad[:EP])


# speedup vs baseline: 1.2361x; 1.2361x over previous
import functools

import jax
import jax.numpy as jnp
from jax import lax
from jax.experimental import pallas as pl
from jax.experimental.pallas import tpu as pltpu
from jax.experimental.pallas import tpu_sc as plsc

N = 10000
NPAD = 10240
E = 320000
EP = 100000
D = 128
H = 128
C = 10

NC = 2
NS = 16
NW = NC * NS
ECH = 128
EROWS = E // ECH
RMAX = 80
PH = 48
ERPAD = 2560
RPS = NPAD // NS
EPPAD = 100352
LCH = EPPAD // NW

_f32 = jnp.float32
_i32 = jnp.int32


def _mesh():
  return plsc.VectorSubcoreMesh(core_axis_name="c", subcore_axis_name="s")


def _tile_rows(wid):
  nrows = jnp.where(wid < 24, 80, jnp.where(wid == 31, 76, 72))
  rbase = jnp.where(wid < 24, wid * 80, 1920 + (wid - 24) * 72)
  return nrows, rbase


def _subcore_rows(s):
  nrows = jnp.where(s < 8, 160, jnp.where(s == 15, 156, 152))
  rbase = jnp.where(s < 8, s * 160, 1280 + (s - 8) * 152)
  return nrows, rbase


def _hist_body(dst_hbm, out_hbm, hist_sh, idx_v, val_v, zrow_v, isem):
  c = lax.axis_index("c")
  s = lax.axis_index("s")
  wid = c * NS + s
  nrows, rbase = _tile_rows(wid)

  icp = pltpu.async_copy(dst_hbm.at[pl.ds(rbase, RMAX)], idx_v, isem)

  ones = jnp.ones((16,), _f32)
  zeros = jnp.zeros((16,), _f32)
  for j in range(ECH // 16):
    val_v[pl.ds(j * 16, 16)] = ones

  def zfill(j, _):
    zrow_v[pl.ds(j * 16, 16)] = zeros
    return 0
  lax.fori_loop(0, RPS // 16, zfill, 0)

  pltpu.sync_copy(zrow_v, hist_sh.at[pl.ds(s * RPS, RPS)])
  icp.wait()
  plsc.subcore_barrier()

  def _drain():
    pltpu.make_async_copy(val_v, hist_sh.at[idx_v.at[0]], isem).wait()

  def chunk(i, _):
    pltpu.async_copy(val_v, hist_sh.at[idx_v.at[i]], isem, add=True)

    @pl.when(i >= 8)
    def _():
      _drain()
    return 0
  lax.fori_loop(0, nrows, chunk, 0)

  def tail(_k, _):
    _drain()
    return 0
  lax.fori_loop(0, jnp.minimum(nrows, 8), tail, 0)

  plsc.subcore_barrier()
  pltpu.sync_copy(hist_sh.at[pl.ds(s * RPS, RPS)], zrow_v)
  pltpu.sync_copy(zrow_v, out_hbm.at[c, pl.ds(s * RPS, RPS)])


def _hist_call(dst2):
  k = pl.kernel(
      _hist_body,
      out_type=jax.ShapeDtypeStruct((NC, NPAD), _f32),
      mesh=_mesh(),
      scratch_types=[
          pltpu.VMEM_SHARED((NPAD,), _f32),
          pltpu.VMEM((RMAX, ECH), _i32),
          pltpu.VMEM((ECH,), _f32),
          pltpu.VMEM((RPS,), _f32),
          pltpu.SemaphoreType.DMA,
      ],
  )
  return k(dst2)


def _seg_body(g_hbm, src_hbm, dst_hbm, out_hbm, s_sh, isrc_v, idst_v,
              rows_a, rows_b, isem, ga_sem, gb_sem, sa_sem, sb_sem):
  c = lax.axis_index("c")
  s = lax.axis_index("s")
  wid = c * NS + s
  nrows, rbase = _tile_rows(wid)

  pltpu.async_copy(src_hbm.at[pl.ds(rbase, PH)], isrc_v, isem)
  pltpu.async_copy(dst_hbm.at[pl.ds(rbase, PH)], idst_v, isem)

  zeros = jnp.zeros((16,), _f32)

  def zfill(t, _):
    rows_a[t // 8, pl.ds((t % 8) * 16, 16)] = zeros
    return 0
  lax.fori_loop(0, ECH * 8, zfill, 0)
  for p in range(RPS // ECH):
    pltpu.sync_copy(rows_a, s_sh.at[pl.ds(s * RPS + p * ECH, ECH)])
  pltpu.make_async_copy(src_hbm.at[pl.ds(rbase, PH)], isrc_v, isem).wait()
  pltpu.make_async_copy(dst_hbm.at[pl.ds(rbase, PH)], idst_v, isem).wait()
  plsc.subcore_barrier()

  def _gather(i, buf, sem):
    pltpu.async_copy(g_hbm.at[isrc_v.at[i]], buf, sem)

  def _scatter(i, buf, sem):
    pltpu.async_copy(buf, s_sh.at[idst_v.at[i]], sem, add=True)

  def _wait(buf, sem):
    pltpu.make_async_copy(g_hbm.at[isrc_v.at[0]], buf, sem).wait()

  def _run_phase(npairs):
    _gather(0, rows_a, ga_sem)

    def pair(j, _):
      i0 = 2 * j
      i1 = i0 + 1
      _gather(i1, rows_b, gb_sem)
      _wait(rows_a, ga_sem)
      pltpu.sync_copy(rows_a, s_sh.at[idst_v.at[i0]], add=True)

      @pl.when(j < npairs - 1)
      def _():
        _gather(i0 + 2, rows_a, ga_sem)
      _wait(rows_b, gb_sem)
      pltpu.sync_copy(rows_b, s_sh.at[idst_v.at[i1]], add=True)
      return 0
    lax.fori_loop(0, npairs, pair, 0)

  _run_phase(PH // 2)
  pltpu.sync_copy(src_hbm.at[pl.ds(rbase + PH, PH)], isrc_v)
  pltpu.sync_copy(dst_hbm.at[pl.ds(rbase + PH, PH)], idst_v)
  _run_phase((nrows - PH) // 2)

  plsc.subcore_barrier()
  for p in range(RPS // ECH):
    r0 = s * RPS + p * ECH
    pltpu.sync_copy(s_sh.at[pl.ds(r0, ECH)], rows_a)
    pltpu.sync_copy(rows_a, out_hbm.at[c, pl.ds(r0, ECH)])


def _seg_call(g, src2, dst2):
  k = pl.kernel(
      _seg_body,
      out_type=jax.ShapeDtypeStruct((NC, NPAD, H), _f32),
      mesh=_mesh(),
      scratch_types=[
          pltpu.VMEM_SHARED((NPAD, H), _f32),
          pltpu.VMEM((PH, ECH), _i32),
          pltpu.VMEM((PH, ECH), _i32),
          pltpu.VMEM((ECH, H), _f32),
          pltpu.VMEM((ECH, H), _f32),
          pltpu.SemaphoreType.DMA,
          pltpu.SemaphoreType.DMA,
          pltpu.SemaphoreType.DMA,
          pltpu.SemaphoreType.DMA,
          pltpu.SemaphoreType.DMA,
      ],
  )
  return k(g, src2, dst2)


def _link_body(u_hbm, v_hbm, ps_hbm, pd_hbm, ns_hbm, nd_hbm,
               lp_hbm, ln_hbm, u_v, v_v, is_v, id_v, ob_v):
  c = lax.axis_index("c")
  s = lax.axis_index("s")
  wid = c * NS + s
  base = wid * LCH

  pltpu.sync_copy(u_hbm, u_v)
  pltpu.sync_copy(v_hbm, v_v)

  for (sh, dh, oh) in ((ps_hbm, pd_hbm, lp_hbm), (ns_hbm, nd_hbm, ln_hbm)):
    pltpu.sync_copy(sh.at[pl.ds(base, LCH)], is_v)
    pltpu.sync_copy(dh.at[pl.ds(base, LCH)], id_v)

    def gath(j, _):
      iu = is_v[pl.ds(j * 16, 16)]
      iv = id_v[pl.ds(j * 16, 16)]
      a = plsc.load_gather(u_v, [iu])
      b = plsc.load_gather(v_v, [iv])
      ob_v[pl.ds(j * 16, 16)] = a + b
      return 0
    lax.fori_loop(0, LCH // 16, gath, 0)
    pltpu.sync_copy(ob_v, oh.at[pl.ds(base, LCH)])


def _link_call(u, v, ps, pd, ns_, nd):
  k = pl.kernel(
      _link_body,
      out_type=[jax.ShapeDtypeStruct((EPPAD,), _f32),
                jax.ShapeDtypeStruct((EPPAD,), _f32)],
      mesh=_mesh(),
      scratch_types=[
          pltpu.VMEM((NPAD,), _f32),
          pltpu.VMEM((NPAD,), _f32),
          pltpu.VMEM((LCH,), _i32),
          pltpu.VMEM((LCH,), _i32),
          pltpu.VMEM((LCH,), _f32),
      ],
      compiler_params=pltpu.CompilerParams(needs_layout_passes=False),
  )
  return k(u, v, ps, pd, ns_, nd)


_BLK = 1024
_GRID = NPAD // _BLK


def _dinv_of(hist_ref):
  return lax.rsqrt(1.0 + hist_ref[0, :] + hist_ref[1, :])[:, None]


def _tc1_body(x_ref, w1_ref, hist_ref, g1_ref):
  h = jnp.dot(x_ref[...], w1_ref[...], preferred_element_type=_f32)
  g1_ref[...] = h * _dinv_of(hist_ref)


def _tc1_call(x_pad, w1, hist):
  return pl.pallas_call(
      _tc1_body,
      grid=(_GRID,),
      in_specs=[
          pl.BlockSpec((_BLK, D), lambda i: (i, 0)),
          pl.BlockSpec((D, H), lambda i: (0, 0)),
          pl.BlockSpec((NC, _BLK), lambda i: (0, i)),
      ],
      out_specs=pl.BlockSpec((_BLK, H), lambda i: (i, 0)),
      out_shape=jax.ShapeDtypeStruct((NPAD, H), _f32),
  )(x_pad, w1, hist)


def _tc2_body(s_ref, g1_ref, hist_ref, b1_ref, w2_ref, g2_ref):
  dinv = _dinv_of(hist_ref)
  o1 = jnp.maximum(
      dinv * (s_ref[0] + s_ref[1] + g1_ref[...]) + b1_ref[...], 0.0)
  h2 = jnp.dot(o1, w2_ref[...], preferred_element_type=_f32)
  g2_ref[...] = h2 * dinv


def _tc2_call(s1, g1, hist, b1, w2):
  return pl.pallas_call(
      _tc2_body,
      grid=(_GRID,),
      in_specs=[
          pl.BlockSpec((NC, _BLK, H), lambda i: (0, i, 0)),
          pl.BlockSpec((_BLK, H), lambda i: (i, 0)),
          pl.BlockSpec((NC, _BLK), lambda i: (0, i)),
          pl.BlockSpec((1, H), lambda i: (0, 0)),
          pl.BlockSpec((H, H), lambda i: (0, 0)),
      ],
      out_specs=pl.BlockSpec((_BLK, H), lambda i: (i, 0)),
      out_shape=jax.ShapeDtypeStruct((NPAD, H), _f32),
  )(s1, g1, hist, b1, w2)


def _tc3_body(s_ref, g2_ref, hist_ref, b2_ref, wc_ref, bc_ref,
              wl_ref, bl_ref, np_ref, u_ref, v_ref):
  dinv = _dinv_of(hist_ref)
  h = dinv * (s_ref[0] + s_ref[1] + g2_ref[...]) + b2_ref[...]
  nh = jnp.maximum(h, 0.0)
  logits = jnp.dot(nh, wc_ref[...], preferred_element_type=_f32) + bc_ref[...]
  m = jnp.max(logits, axis=1, keepdims=True)
  lse = jnp.log(jnp.sum(jnp.exp(logits - m), axis=1, keepdims=True)) + m
  np_ref[...] = logits - lse
  uv = jnp.dot(h, wl_ref[...], preferred_element_type=_f32)
  u_ref[...] = uv[:, 0:1] + bl_ref[...]
  v_ref[...] = uv[:, 1:2]


def _tc3_call(s2, g2, hist, b2, wc, bc, wl2, bl):
  return pl.pallas_call(
      _tc3_body,
      grid=(_GRID,),
      in_specs=[
          pl.BlockSpec((NC, _BLK, H), lambda i: (0, i, 0)),
          pl.BlockSpec((_BLK, H), lambda i: (i, 0)),
          pl.BlockSpec((NC, _BLK), lambda i: (0, i)),
          pl.BlockSpec((1, H), lambda i: (0, 0)),
          pl.BlockSpec((H, C), lambda i: (0, 0)),
          pl.BlockSpec((1, C), lambda i: (0, 0)),
          pl.BlockSpec((H, 2), lambda i: (0, 0)),
          pl.BlockSpec((1, 1), lambda i: (0, 0)),
      ],
      out_specs=[
          pl.BlockSpec((_BLK, C), lambda i: (i, 0)),
          pl.BlockSpec((_BLK, 1), lambda i: (i, 0)),
          pl.BlockSpec((_BLK, 1), lambda i: (i, 0)),
      ],
      out_shape=[
          jax.ShapeDtypeStruct((NPAD, C), _f32),
          jax.ShapeDtypeStruct((NPAD, 1), _f32),
          jax.ShapeDtypeStruct((NPAD, 1), _f32),
      ],
  )(s2, g2, hist, b2, wc, bc, wl2, bl)


def kernel(x, edge_index, edge_index_pos, edge_index_neg,
           W1, b1, W2, b2, Wc, bc, Wl, bl):
  src2 = jnp.pad(edge_index[0].reshape(EROWS, ECH),
                 ((0, ERPAD - EROWS), (0, 0)))
  dst2 = jnp.pad(edge_index[1].reshape(EROWS, ECH),
                 ((0, ERPAD - EROWS), (0, 0)))
  x_pad = jnp.pad(x, ((0, NPAD - N), (0, 0)))

  hist = _hist_call(dst2)

  g1 = _tc1_call(x_pad, W1, hist)
  s1 = _seg_call(g1, src2, dst2)
  g2 = _tc2_call(s1, g1, hist, b1.reshape(1, H), W2)
  s2 = _seg_call(g2, src2, dst2)

  wl2 = jnp.concatenate([Wl[:H], Wl[H:]], axis=1)
  node_pad, u2, v2 = _tc3_call(s2, g2, hist, b2.reshape(1, H), Wc,
                               bc.reshape(1, C), wl2, bl.reshape(1, 1))

  pad = (0, EPPAD - EP)
  lp_pad, ln_pad = _link_call(
      u2[:, 0], v2[:, 0],
      jnp.pad(edge_index_pos[0], pad), jnp.pad(edge_index_pos[1], pad),
      jnp.pad(edge_index_neg[0], pad), jnp.pad(edge_index_neg[1], pad))

  return (node_pad[:N], lp_pad[:EP], ln_pad[:EP])


# speedup vs baseline: 1.2471x; 1.0089x over previous
import functools

import jax
import jax.numpy as jnp
from jax import lax
from jax.experimental import pallas as pl
from jax.experimental.pallas import tpu as pltpu
from jax.experimental.pallas import tpu_sc as plsc

N = 10000
NPAD = 10240
E = 320000
EP = 100000
D = 128
H = 128
C = 10

NC = 2
NS = 16
NW = NC * NS
ECH = 128
EROWS = E // ECH
RMAX = 80
PH = 48
ERPAD = 2560
RPS = NPAD // NS
EPPAD = 100352
LCH = EPPAD // NW

_f32 = jnp.float32
_i32 = jnp.int32


def _mesh():
  return plsc.VectorSubcoreMesh(core_axis_name="c", subcore_axis_name="s")


def _tile_rows(wid):
  nrows = jnp.where(wid < 24, 80, jnp.where(wid == 31, 76, 72))
  rbase = jnp.where(wid < 24, wid * 80, 1920 + (wid - 24) * 72)
  return nrows, rbase


def _subcore_rows(s):
  nrows = jnp.where(s < 8, 160, jnp.where(s == 15, 156, 152))
  rbase = jnp.where(s < 8, s * 160, 1280 + (s - 8) * 152)
  return nrows, rbase


def _hist_body(dst_hbm, out_hbm, hist_sh, idx_v, val_v, zrow_v, isem):
  c = lax.axis_index("c")
  s = lax.axis_index("s")
  wid = c * NS + s
  nrows, rbase = _tile_rows(wid)

  icp = pltpu.async_copy(dst_hbm.at[pl.ds(rbase, RMAX)], idx_v, isem)

  ones = jnp.ones((16,), _f32)
  zeros = jnp.zeros((16,), _f32)
  for j in range(ECH // 16):
    val_v[pl.ds(j * 16, 16)] = ones

  def zfill(j, _):
    zrow_v[pl.ds(j * 16, 16)] = zeros
    return 0
  lax.fori_loop(0, RPS // 16, zfill, 0)

  pltpu.sync_copy(zrow_v, hist_sh.at[pl.ds(s * RPS, RPS)])
  icp.wait()
  plsc.subcore_barrier()

  def _drain():
    pltpu.make_async_copy(val_v, hist_sh.at[idx_v.at[0]], isem).wait()

  def chunk(i, _):
    pltpu.async_copy(val_v, hist_sh.at[idx_v.at[i]], isem, add=True)

    @pl.when(i >= 8)
    def _():
      _drain()
    return 0
  lax.fori_loop(0, nrows, chunk, 0)

  def tail(_k, _):
    _drain()
    return 0
  lax.fori_loop(0, jnp.minimum(nrows, 8), tail, 0)

  plsc.subcore_barrier()
  pltpu.sync_copy(hist_sh.at[pl.ds(s * RPS, RPS)], zrow_v)
  pltpu.sync_copy(zrow_v, out_hbm.at[c, pl.ds(s * RPS, RPS)])


def _hist_call(dst2):
  k = pl.kernel(
      _hist_body,
      out_type=jax.ShapeDtypeStruct((NC, NPAD), _f32),
      mesh=_mesh(),
      scratch_types=[
          pltpu.VMEM_SHARED((NPAD,), _f32),
          pltpu.VMEM((RMAX, ECH), _i32),
          pltpu.VMEM((ECH,), _f32),
          pltpu.VMEM((RPS,), _f32),
          pltpu.SemaphoreType.DMA,
      ],
  )
  return k(dst2)


def _seg_body(g_hbm, src_hbm, dst_hbm, out_hbm, s_sh, isrc_v, idst_v,
              rows_a, rows_b, isem, ga_sem, gb_sem, sa_sem, sb_sem):
  c = lax.axis_index("c")
  s = lax.axis_index("s")
  wid = c * NS + s
  nrows, rbase = _tile_rows(wid)

  pltpu.async_copy(src_hbm.at[pl.ds(rbase, PH)], isrc_v, isem)
  pltpu.async_copy(dst_hbm.at[pl.ds(rbase, PH)], idst_v, isem)

  zeros = jnp.zeros((16,), _f32)

  def zfill(t, _):
    rows_a[t // 8, pl.ds((t % 8) * 16, 16)] = zeros
    return 0
  lax.fori_loop(0, ECH * 8, zfill, 0)
  for p in range(RPS // ECH):
    pltpu.sync_copy(rows_a, s_sh.at[pl.ds(s * RPS + p * ECH, ECH)])
  pltpu.make_async_copy(src_hbm.at[pl.ds(rbase, PH)], isrc_v, isem).wait()
  pltpu.make_async_copy(dst_hbm.at[pl.ds(rbase, PH)], idst_v, isem).wait()
  plsc.subcore_barrier()

  def _gather(i, buf, sem):
    pltpu.async_copy(g_hbm.at[isrc_v.at[i]], buf, sem)

  def _scatter(i, buf, sem):
    pltpu.async_copy(buf, s_sh.at[idst_v.at[i]], sem, add=True)

  def _wait(buf, sem):
    pltpu.make_async_copy(g_hbm.at[isrc_v.at[0]], buf, sem).wait()

  def _run_phase(npairs):
    _gather(0, rows_a, ga_sem)

    def pair(j, _):
      i0 = 2 * j
      i1 = i0 + 1
      _gather(i1, rows_b, gb_sem)
      _wait(rows_a, ga_sem)
      pltpu.sync_copy(rows_a, s_sh.at[idst_v.at[i0]], add=True)

      @pl.when(j < npairs - 1)
      def _():
        _gather(i0 + 2, rows_a, ga_sem)
      _wait(rows_b, gb_sem)
      pltpu.sync_copy(rows_b, s_sh.at[idst_v.at[i1]], add=True)
      return 0
    lax.fori_loop(0, npairs, pair, 0)

  _run_phase(PH // 2)
  pltpu.sync_copy(src_hbm.at[pl.ds(rbase + PH, PH)], isrc_v)
  pltpu.sync_copy(dst_hbm.at[pl.ds(rbase + PH, PH)], idst_v)
  _run_phase((nrows - PH) // 2)

  plsc.subcore_barrier()
  rb = s * RPS
  obufs = ((rows_a, sa_sem), (rows_b, sb_sem))
  npieces = RPS // ECH
  for p in range(npieces):
    buf, ssem = obufs[p % 2]
    if p >= 2:
      pltpu.make_async_copy(buf, out_hbm.at[c, pl.ds(rb, ECH)], ssem).wait()
    pltpu.sync_copy(s_sh.at[pl.ds(rb + p * ECH, ECH)], buf)
    pltpu.async_copy(buf, out_hbm.at[c, pl.ds(rb + p * ECH, ECH)], ssem)
  for p in (npieces - 2, npieces - 1):
    buf, ssem = obufs[p % 2]
    pltpu.make_async_copy(buf, out_hbm.at[c, pl.ds(rb, ECH)], ssem).wait()


def _seg_call(g, src2, dst2):
  k = pl.kernel(
      _seg_body,
      out_type=jax.ShapeDtypeStruct((NC, NPAD, H), _f32),
      mesh=_mesh(),
      scratch_types=[
          pltpu.VMEM_SHARED((NPAD, H), _f32),
          pltpu.VMEM((PH, ECH), _i32),
          pltpu.VMEM((PH, ECH), _i32),
          pltpu.VMEM((ECH, H), _f32),
          pltpu.VMEM((ECH, H), _f32),
          pltpu.SemaphoreType.DMA,
          pltpu.SemaphoreType.DMA,
          pltpu.SemaphoreType.DMA,
          pltpu.SemaphoreType.DMA,
          pltpu.SemaphoreType.DMA,
      ],
  )
  return k(g, src2, dst2)


def _link_body(u_hbm, v_hbm, ps_hbm, pd_hbm, ns_hbm, nd_hbm,
               lp_hbm, ln_hbm, u_v, v_v, is_v, id_v, ob_v):
  c = lax.axis_index("c")
  s = lax.axis_index("s")
  wid = c * NS + s
  base = wid * LCH

  pltpu.sync_copy(u_hbm, u_v)
  pltpu.sync_copy(v_hbm, v_v)

  for (sh, dh, oh) in ((ps_hbm, pd_hbm, lp_hbm), (ns_hbm, nd_hbm, ln_hbm)):
    pltpu.sync_copy(sh.at[pl.ds(base, LCH)], is_v)
    pltpu.sync_copy(dh.at[pl.ds(base, LCH)], id_v)

    def gath(j, _):
      iu = is_v[pl.ds(j * 16, 16)]
      iv = id_v[pl.ds(j * 16, 16)]
      a = plsc.load_gather(u_v, [iu])
      b = plsc.load_gather(v_v, [iv])
      ob_v[pl.ds(j * 16, 16)] = a + b
      return 0
    lax.fori_loop(0, LCH // 16, gath, 0)
    pltpu.sync_copy(ob_v, oh.at[pl.ds(base, LCH)])


def _link_call(u, v, ps, pd, ns_, nd):
  k = pl.kernel(
      _link_body,
      out_type=[jax.ShapeDtypeStruct((EPPAD,), _f32),
                jax.ShapeDtypeStruct((EPPAD,), _f32)],
      mesh=_mesh(),
      scratch_types=[
          pltpu.VMEM((NPAD,), _f32),
          pltpu.VMEM((NPAD,), _f32),
          pltpu.VMEM((LCH,), _i32),
          pltpu.VMEM((LCH,), _i32),
          pltpu.VMEM((LCH,), _f32),
      ],
      compiler_params=pltpu.CompilerParams(needs_layout_passes=False),
  )
  return k(u, v, ps, pd, ns_, nd)


_BLK = 1024
_GRID = NPAD // _BLK


def _dinv_of(hist_ref):
  return lax.rsqrt(1.0 + hist_ref[0, :] + hist_ref[1, :])[:, None]


def _tc1_body(x_ref, w1_ref, hist_ref, g1_ref):
  h = jnp.dot(x_ref[...], w1_ref[...], preferred_element_type=_f32)
  g1_ref[...] = h * _dinv_of(hist_ref)


def _tc1_call(x_pad, w1, hist):
  return pl.pallas_call(
      _tc1_body,
      grid=(_GRID,),
      in_specs=[
          pl.BlockSpec((_BLK, D), lambda i: (i, 0)),
          pl.BlockSpec((D, H), lambda i: (0, 0)),
          pl.BlockSpec((NC, _BLK), lambda i: (0, i)),
      ],
      out_specs=pl.BlockSpec((_BLK, H), lambda i: (i, 0)),
      out_shape=jax.ShapeDtypeStruct((NPAD, H), _f32),
  )(x_pad, w1, hist)


def _tc2_body(s_ref, g1_ref, hist_ref, b1_ref, w2_ref, g2_ref):
  dinv = _dinv_of(hist_ref)
  o1 = jnp.maximum(
      dinv * (s_ref[0] + s_ref[1] + g1_ref[...]) + b1_ref[...], 0.0)
  h2 = jnp.dot(o1, w2_ref[...], preferred_element_type=_f32)
  g2_ref[...] = h2 * dinv


def _tc2_call(s1, g1, hist, b1, w2):
  return pl.pallas_call(
      _tc2_body,
      grid=(_GRID,),
      in_specs=[
          pl.BlockSpec((NC, _BLK, H), lambda i: (0, i, 0)),
          pl.BlockSpec((_BLK, H), lambda i: (i, 0)),
          pl.BlockSpec((NC, _BLK), lambda i: (0, i)),
          pl.BlockSpec((1, H), lambda i: (0, 0)),
          pl.BlockSpec((H, H), lambda i: (0, 0)),
      ],
      out_specs=pl.BlockSpec((_BLK, H), lambda i: (i, 0)),
      out_shape=jax.ShapeDtypeStruct((NPAD, H), _f32),
  )(s1, g1, hist, b1, w2)


def _tc3_body(s_ref, g2_ref, hist_ref, b2_ref, wc_ref, bc_ref,
              wl_ref, bl_ref, np_ref, u_ref, v_ref):
  dinv = _dinv_of(hist_ref)
  h = dinv * (s_ref[0] + s_ref[1] + g2_ref[...]) + b2_ref[...]
  nh = jnp.maximum(h, 0.0)
  logits = jnp.dot(nh, wc_ref[...], preferred_element_type=_f32) + bc_ref[...]
  m = jnp.max(logits, axis=1, keepdims=True)
  lse = jnp.log(jnp.sum(jnp.exp(logits - m), axis=1, keepdims=True)) + m
  np_ref[...] = logits - lse
  uv = jnp.dot(h, wl_ref[...], preferred_element_type=_f32)
  u_ref[...] = uv[:, 0:1] + bl_ref[...]
  v_ref[...] = uv[:, 1:2]


def _tc3_call(s2, g2, hist, b2, wc, bc, wl2, bl):
  return pl.pallas_call(
      _tc3_body,
      grid=(_GRID,),
      in_specs=[
          pl.BlockSpec((NC, _BLK, H), lambda i: (0, i, 0)),
          pl.BlockSpec((_BLK, H), lambda i: (i, 0)),
          pl.BlockSpec((NC, _BLK), lambda i: (0, i)),
          pl.BlockSpec((1, H), lambda i: (0, 0)),
          pl.BlockSpec((H, C), lambda i: (0, 0)),
          pl.BlockSpec((1, C), lambda i: (0, 0)),
          pl.BlockSpec((H, 2), lambda i: (0, 0)),
          pl.BlockSpec((1, 1), lambda i: (0, 0)),
      ],
      out_specs=[
          pl.BlockSpec((_BLK, C), lambda i: (i, 0)),
          pl.BlockSpec((_BLK, 1), lambda i: (i, 0)),
          pl.BlockSpec((_BLK, 1), lambda i: (i, 0)),
      ],
      out_shape=[
          jax.ShapeDtypeStruct((NPAD, C), _f32),
          jax.ShapeDtypeStruct((NPAD, 1), _f32),
          jax.ShapeDtypeStruct((NPAD, 1), _f32),
      ],
  )(s2, g2, hist, b2, wc, bc, wl2, bl)


def kernel(x, edge_index, edge_index_pos, edge_index_neg,
           W1, b1, W2, b2, Wc, bc, Wl, bl):
  src2 = jnp.pad(edge_index[0].reshape(EROWS, ECH),
                 ((0, ERPAD - EROWS), (0, 0)))
  dst2 = jnp.pad(edge_index[1].reshape(EROWS, ECH),
                 ((0, ERPAD - EROWS), (0, 0)))
  x_pad = jnp.pad(x, ((0, NPAD - N), (0, 0)))

  hist = _hist_call(dst2)

  g1 = _tc1_call(x_pad, W1, hist)
  s1 = _seg_call(g1, src2, dst2)
  g2 = _tc2_call(s1, g1, hist, b1.reshape(1, H), W2)
  s2 = _seg_call(g2, src2, dst2)

  wl2 = jnp.concatenate([Wl[:H], Wl[H:]], axis=1)
  node_pad, u2, v2 = _tc3_call(s2, g2, hist, b2.reshape(1, H), Wc,
                               bc.reshape(1, C), wl2, bl.reshape(1, 1))

  pad = (0, EPPAD - EP)
  lp_pad, ln_pad = _link_call(
      u2[:, 0], v2[:, 0],
      jnp.pad(edge_index_pos[0], pad), jnp.pad(edge_index_pos[1], pad),
      jnp.pad(edge_index_neg[0], pad), jnp.pad(edge_index_neg[1], pad))

  return (node_pad[:N], lp_pad[:EP], ln_pad[:EP])


# speedup vs baseline: 1.2541x; 1.0056x over previous
import functools

import jax
import jax.numpy as jnp
from jax import lax
from jax.experimental import pallas as pl
from jax.experimental.pallas import tpu as pltpu
from jax.experimental.pallas import tpu_sc as plsc

N = 10000
NPAD = 10240
E = 320000
EP = 100000
D = 128
H = 128
C = 10

NC = 2
NS = 16
NW = NC * NS
ECH = 128
EROWS = E // ECH
RMAX = 80
PH = 48
ERPAD = 2560
RPS = NPAD // NS
EPPAD = 100352
LCH = EPPAD // NW

_f32 = jnp.float32
_i32 = jnp.int32


def _mesh():
  return plsc.VectorSubcoreMesh(core_axis_name="c", subcore_axis_name="s")


def _tile_rows(wid):
  nrows = jnp.where(wid < 24, 80, jnp.where(wid == 31, 76, 72))
  rbase = jnp.where(wid < 24, wid * 80, 1920 + (wid - 24) * 72)
  return nrows, rbase


def _subcore_rows(s):
  nrows = jnp.where(s < 8, 160, jnp.where(s == 15, 156, 152))
  rbase = jnp.where(s < 8, s * 160, 1280 + (s - 8) * 152)
  return nrows, rbase


def _hist_body(dst_hbm, out_hbm, hist_sh, idx_v, val_v, zrow_v, isem):
  c = lax.axis_index("c")
  s = lax.axis_index("s")
  wid = c * NS + s
  nrows, rbase = _tile_rows(wid)

  icp = pltpu.async_copy(dst_hbm.at[pl.ds(rbase, RMAX)], idx_v, isem)

  ones = jnp.ones((16,), _f32)
  zeros = jnp.zeros((16,), _f32)
  for j in range(ECH // 16):
    val_v[pl.ds(j * 16, 16)] = ones

  def zfill(j, _):
    zrow_v[pl.ds(j * 16, 16)] = zeros
    return 0
  lax.fori_loop(0, RPS // 16, zfill, 0)

  pltpu.sync_copy(zrow_v, hist_sh.at[pl.ds(s * RPS, RPS)])
  icp.wait()
  plsc.subcore_barrier()

  def _drain():
    pltpu.make_async_copy(val_v, hist_sh.at[idx_v.at[0]], isem).wait()

  def chunk(i, _):
    pltpu.async_copy(val_v, hist_sh.at[idx_v.at[i]], isem, add=True)

    @pl.when(i >= 8)
    def _():
      _drain()
    return 0
  lax.fori_loop(0, nrows, chunk, 0)

  def tail(_k, _):
    _drain()
    return 0
  lax.fori_loop(0, jnp.minimum(nrows, 8), tail, 0)

  plsc.subcore_barrier()
  pltpu.sync_copy(hist_sh.at[pl.ds(s * RPS, RPS)], zrow_v)
  pltpu.sync_copy(zrow_v, out_hbm.at[c, pl.ds(s * RPS, RPS)])


def _hist_call(dst2):
  k = pl.kernel(
      _hist_body,
      out_type=jax.ShapeDtypeStruct((NC, NPAD), _f32),
      mesh=_mesh(),
      scratch_types=[
          pltpu.VMEM_SHARED((NPAD,), _f32),
          pltpu.VMEM((RMAX, ECH), _i32),
          pltpu.VMEM((ECH,), _f32),
          pltpu.VMEM((RPS,), _f32),
          pltpu.SemaphoreType.DMA,
      ],
  )
  return k(dst2)


def _seg_body(g_hbm, src_hbm, dst_hbm, out_hbm, s_sh, isrc_v, idst_v,
              rows_a, rows_b, isem, ga_sem, gb_sem, sa_sem, sb_sem):
  c = lax.axis_index("c")
  s = lax.axis_index("s")
  wid = c * NS + s
  nrows, rbase = _tile_rows(wid)

  pltpu.async_copy(src_hbm.at[pl.ds(rbase, PH)], isrc_v, isem)
  pltpu.async_copy(dst_hbm.at[pl.ds(rbase, PH)], idst_v, isem)

  zeros = jnp.zeros((16,), _f32)

  def zfill(t, _):
    rows_a[t // 8, pl.ds((t % 8) * 16, 16)] = zeros
    return 0
  lax.fori_loop(0, ECH * 8, zfill, 0)
  for p in range(RPS // ECH):
    pltpu.sync_copy(rows_a, s_sh.at[pl.ds(s * RPS + p * ECH, ECH)])
  pltpu.make_async_copy(src_hbm.at[pl.ds(rbase, PH)], isrc_v, isem).wait()
  pltpu.make_async_copy(dst_hbm.at[pl.ds(rbase, PH)], idst_v, isem).wait()
  plsc.subcore_barrier()

  def _gather(i, buf, sem):
    pltpu.async_copy(g_hbm.at[isrc_v.at[i]], buf, sem)

  def _scatter(i, buf, sem):
    pltpu.async_copy(buf, s_sh.at[idst_v.at[i]], sem, add=True)

  def _wait(buf, sem):
    pltpu.make_async_copy(g_hbm.at[isrc_v.at[0]], buf, sem).wait()

  def _run_phase(npairs):
    _gather(0, rows_a, ga_sem)

    def pair(j, _):
      i0 = 2 * j
      i1 = i0 + 1
      _gather(i1, rows_b, gb_sem)
      _wait(rows_a, ga_sem)
      pltpu.sync_copy(rows_a, s_sh.at[idst_v.at[i0]], add=True)

      @pl.when(j < npairs - 1)
      def _():
        _gather(i0 + 2, rows_a, ga_sem)
      _wait(rows_b, gb_sem)
      pltpu.sync_copy(rows_b, s_sh.at[idst_v.at[i1]], add=True)
      return 0
    lax.fori_loop(0, npairs, pair, 0)

  _run_phase(PH // 2)
  pltpu.sync_copy(src_hbm.at[pl.ds(rbase + PH, PH)], isrc_v)
  pltpu.sync_copy(dst_hbm.at[pl.ds(rbase + PH, PH)], idst_v)
  _run_phase((nrows - PH) // 2)

  plsc.subcore_barrier()
  rb = s * RPS
  obufs = ((rows_a, sa_sem), (rows_b, sb_sem))
  npieces = RPS // ECH
  for p in range(npieces):
    buf, ssem = obufs[p % 2]
    if p >= 2:
      pltpu.make_async_copy(buf, out_hbm.at[c, pl.ds(rb, ECH)], ssem).wait()
    pltpu.sync_copy(s_sh.at[pl.ds(rb + p * ECH, ECH)], buf)
    pltpu.async_copy(buf, out_hbm.at[c, pl.ds(rb + p * ECH, ECH)], ssem)
  for p in (npieces - 2, npieces - 1):
    buf, ssem = obufs[p % 2]
    pltpu.make_async_copy(buf, out_hbm.at[c, pl.ds(rb, ECH)], ssem).wait()


def _seg_call(g, src2, dst2):
  k = pl.kernel(
      _seg_body,
      out_type=jax.ShapeDtypeStruct((NC, NPAD, H), _f32),
      mesh=_mesh(),
      scratch_types=[
          pltpu.VMEM_SHARED((NPAD, H), _f32),
          pltpu.VMEM((PH, ECH), _i32),
          pltpu.VMEM((PH, ECH), _i32),
          pltpu.VMEM((ECH, H), _f32),
          pltpu.VMEM((ECH, H), _f32),
          pltpu.SemaphoreType.DMA,
          pltpu.SemaphoreType.DMA,
          pltpu.SemaphoreType.DMA,
          pltpu.SemaphoreType.DMA,
          pltpu.SemaphoreType.DMA,
      ],
  )
  return k(g, src2, dst2)


def _link_body(u_hbm, v_hbm, ps_hbm, pd_hbm, ns_hbm, nd_hbm,
               lp_hbm, ln_hbm, u_v, v_v, is0_v, id0_v, is1_v, id1_v,
               ob_v, ob2_v, lsem):
  c = lax.axis_index("c")
  s = lax.axis_index("s")
  wid = c * NS + s
  base = wid * LCH

  pltpu.async_copy(u_hbm, u_v, lsem)
  pltpu.async_copy(v_hbm, v_v, lsem)
  pltpu.async_copy(ps_hbm.at[pl.ds(base, LCH)], is0_v, lsem)
  pltpu.async_copy(pd_hbm.at[pl.ds(base, LCH)], id0_v, lsem)
  pltpu.async_copy(ns_hbm.at[pl.ds(base, LCH)], is1_v, lsem)
  pltpu.async_copy(nd_hbm.at[pl.ds(base, LCH)], id1_v, lsem)
  for _ in range(2):
    pltpu.make_async_copy(u_hbm, u_v, lsem).wait()
  for _ in range(4):
    pltpu.make_async_copy(ps_hbm.at[pl.ds(base, LCH)], is0_v, lsem).wait()

  for iss, ids, oh, ob in ((is0_v, id0_v, lp_hbm, ob_v),
                           (is1_v, id1_v, ln_hbm, ob2_v)):
    def gath(j, _, iss=iss, ids=ids, ob=ob):
      for t in range(4):
        o = (4 * j + t) * 16
        a = plsc.load_gather(u_v, [iss[pl.ds(o, 16)]])
        b = plsc.load_gather(v_v, [ids[pl.ds(o, 16)]])
        ob[pl.ds(o, 16)] = a + b
      return 0
    lax.fori_loop(0, LCH // 64, gath, 0)
    pltpu.async_copy(ob, oh.at[pl.ds(base, LCH)], lsem)
  for _ in range(2):
    pltpu.make_async_copy(ob_v, lp_hbm.at[pl.ds(base, LCH)], lsem).wait()


def _link_call(u, v, ps, pd, ns_, nd):
  k = pl.kernel(
      _link_body,
      out_type=[jax.ShapeDtypeStruct((EPPAD,), _f32),
                jax.ShapeDtypeStruct((EPPAD,), _f32)],
      mesh=_mesh(),
      scratch_types=[
          pltpu.VMEM((NPAD,), _f32),
          pltpu.VMEM((NPAD,), _f32),
          pltpu.VMEM((LCH,), _i32),
          pltpu.VMEM((LCH,), _i32),
          pltpu.VMEM((LCH,), _i32),
          pltpu.VMEM((LCH,), _i32),
          pltpu.VMEM((LCH,), _f32),
          pltpu.VMEM((LCH,), _f32),
          pltpu.SemaphoreType.DMA,
      ],
      compiler_params=pltpu.CompilerParams(needs_layout_passes=False),
  )
  return k(u, v, ps, pd, ns_, nd)


_BLK = 1024
_GRID = NPAD // _BLK


def _dinv_of(hist_ref):
  return lax.rsqrt(1.0 + hist_ref[0, :] + hist_ref[1, :])[:, None]


def _tc1_body(x_ref, w1_ref, hist_ref, g1_ref):
  h = jnp.dot(x_ref[...], w1_ref[...], preferred_element_type=_f32)
  g1_ref[...] = h * _dinv_of(hist_ref)


def _tc1_call(x_pad, w1, hist):
  return pl.pallas_call(
      _tc1_body,
      grid=(_GRID,),
      in_specs=[
          pl.BlockSpec((_BLK, D), lambda i: (i, 0)),
          pl.BlockSpec((D, H), lambda i: (0, 0)),
          pl.BlockSpec((NC, _BLK), lambda i: (0, i)),
      ],
      out_specs=pl.BlockSpec((_BLK, H), lambda i: (i, 0)),
      out_shape=jax.ShapeDtypeStruct((NPAD, H), _f32),
  )(x_pad, w1, hist)


def _tc2_body(s_ref, g1_ref, hist_ref, b1_ref, w2_ref, g2_ref):
  dinv = _dinv_of(hist_ref)
  o1 = jnp.maximum(
      dinv * (s_ref[0] + s_ref[1] + g1_ref[...]) + b1_ref[...], 0.0)
  h2 = jnp.dot(o1, w2_ref[...], preferred_element_type=_f32)
  g2_ref[...] = h2 * dinv


def _tc2_call(s1, g1, hist, b1, w2):
  return pl.pallas_call(
      _tc2_body,
      grid=(_GRID,),
      in_specs=[
          pl.BlockSpec((NC, _BLK, H), lambda i: (0, i, 0)),
          pl.BlockSpec((_BLK, H), lambda i: (i, 0)),
          pl.BlockSpec((NC, _BLK), lambda i: (0, i)),
          pl.BlockSpec((1, H), lambda i: (0, 0)),
          pl.BlockSpec((H, H), lambda i: (0, 0)),
      ],
      out_specs=pl.BlockSpec((_BLK, H), lambda i: (i, 0)),
      out_shape=jax.ShapeDtypeStruct((NPAD, H), _f32),
  )(s1, g1, hist, b1, w2)


def _tc3_body(s_ref, g2_ref, hist_ref, b2_ref, wc_ref, bc_ref,
              wl_ref, bl_ref, np_ref, u_ref, v_ref):
  dinv = _dinv_of(hist_ref)
  h = dinv * (s_ref[0] + s_ref[1] + g2_ref[...]) + b2_ref[...]
  nh = jnp.maximum(h, 0.0)
  logits = jnp.dot(nh, wc_ref[...], preferred_element_type=_f32) + bc_ref[...]
  m = jnp.max(logits, axis=1, keepdims=True)
  lse = jnp.log(jnp.sum(jnp.exp(logits - m), axis=1, keepdims=True)) + m
  np_ref[...] = logits - lse
  uv = jnp.dot(h, wl_ref[...], preferred_element_type=_f32)
  u_ref[...] = uv[:, 0:1] + bl_ref[...]
  v_ref[...] = uv[:, 1:2]


def _tc3_call(s2, g2, hist, b2, wc, bc, wl2, bl):
  return pl.pallas_call(
      _tc3_body,
      grid=(_GRID,),
      in_specs=[
          pl.BlockSpec((NC, _BLK, H), lambda i: (0, i, 0)),
          pl.BlockSpec((_BLK, H), lambda i: (i, 0)),
          pl.BlockSpec((NC, _BLK), lambda i: (0, i)),
          pl.BlockSpec((1, H), lambda i: (0, 0)),
          pl.BlockSpec((H, C), lambda i: (0, 0)),
          pl.BlockSpec((1, C), lambda i: (0, 0)),
          pl.BlockSpec((H, 2), lambda i: (0, 0)),
          pl.BlockSpec((1, 1), lambda i: (0, 0)),
      ],
      out_specs=[
          pl.BlockSpec((_BLK, C), lambda i: (i, 0)),
          pl.BlockSpec((_BLK, 1), lambda i: (i, 0)),
          pl.BlockSpec((_BLK, 1), lambda i: (i, 0)),
      ],
      out_shape=[
          jax.ShapeDtypeStruct((NPAD, C), _f32),
          jax.ShapeDtypeStruct((NPAD, 1), _f32),
          jax.ShapeDtypeStruct((NPAD, 1), _f32),
      ],
  )(s2, g2, hist, b2, wc, bc, wl2, bl)


def kernel(x, edge_index, edge_index_pos, edge_index_neg,
           W1, b1, W2, b2, Wc, bc, Wl, bl):
  src2 = jnp.pad(edge_index[0].reshape(EROWS, ECH),
                 ((0, ERPAD - EROWS), (0, 0)))
  dst2 = jnp.pad(edge_index[1].reshape(EROWS, ECH),
                 ((0, ERPAD - EROWS), (0, 0)))
  x_pad = jnp.pad(x, ((0, NPAD - N), (0, 0)))

  hist = _hist_call(dst2)

  g1 = _tc1_call(x_pad, W1, hist)
  s1 = _seg_call(g1, src2, dst2)
  g2 = _tc2_call(s1, g1, hist, b1.reshape(1, H), W2)
  s2 = _seg_call(g2, src2, dst2)

  wl2 = jnp.concatenate([Wl[:H], Wl[H:]], axis=1)
  node_pad, u2, v2 = _tc3_call(s2, g2, hist, b2.reshape(1, H), Wc,
                               bc.reshape(1, C), wl2, bl.reshape(1, 1))

  pad = (0, EPPAD - EP)
  lp_pad, ln_pad = _link_call(
      u2[:, 0], v2[:, 0],
      jnp.pad(edge_index_pos[0], pad), jnp.pad(edge_index_pos[1], pad),
      jnp.pad(edge_index_neg[0], pad), jnp.pad(edge_index_neg[1], pad))

  return (node_pad[:N], lp_pad[:EP], ln_pad[:EP])


# speedup vs baseline: 1.2803x; 1.0209x over previous
import functools

import jax
import jax.numpy as jnp
from jax import lax
from jax.experimental import pallas as pl
from jax.experimental.pallas import tpu as pltpu
from jax.experimental.pallas import tpu_sc as plsc

N = 10000
NPAD = 10240
E = 320000
EP = 100000
D = 128
H = 128
C = 10

NC = 2
NS = 16
NW = NC * NS
ECH = 128
EROWS = E // ECH
RMAX = 80
PH = 48
ERPAD = 2560
RPS = NPAD // NS
EPPAD = 100352
LCH = EPPAD // NW

_f32 = jnp.float32
_i32 = jnp.int32


def _mesh():
  return plsc.VectorSubcoreMesh(core_axis_name="c", subcore_axis_name="s")


def _tile_rows(wid):
  nrows = jnp.where(wid < 24, 80, jnp.where(wid == 31, 76, 72))
  rbase = jnp.where(wid < 24, wid * 80, 1920 + (wid - 24) * 72)
  return nrows, rbase


def _subcore_rows(s):
  nrows = jnp.where(s < 8, 160, jnp.where(s == 15, 156, 152))
  rbase = jnp.where(s < 8, s * 160, 1280 + (s - 8) * 152)
  return nrows, rbase


def _hist_body(dst_hbm, out_hbm, hist_sh, idx_v, val_v, zrow_v, isem):
  c = lax.axis_index("c")
  s = lax.axis_index("s")
  wid = c * NS + s
  nrows, rbase = _tile_rows(wid)

  icp = pltpu.async_copy(dst_hbm.at[pl.ds(rbase, RMAX)], idx_v, isem)

  ones = jnp.ones((16,), _f32)
  zeros = jnp.zeros((16,), _f32)
  for j in range(ECH // 16):
    val_v[pl.ds(j * 16, 16)] = ones

  def zfill(j, _):
    zrow_v[pl.ds(j * 16, 16)] = zeros
    return 0
  lax.fori_loop(0, RPS // 16, zfill, 0)

  pltpu.sync_copy(zrow_v, hist_sh.at[pl.ds(s * RPS, RPS)])
  icp.wait()
  plsc.subcore_barrier()

  def _drain():
    pltpu.make_async_copy(val_v, hist_sh.at[idx_v.at[0]], isem).wait()

  def chunk(i, _):
    pltpu.async_copy(val_v, hist_sh.at[idx_v.at[i]], isem, add=True)

    @pl.when(i >= 8)
    def _():
      _drain()
    return 0
  lax.fori_loop(0, nrows, chunk, 0)

  def tail(_k, _):
    _drain()
    return 0
  lax.fori_loop(0, jnp.minimum(nrows, 8), tail, 0)

  plsc.subcore_barrier()
  pltpu.sync_copy(hist_sh.at[pl.ds(s * RPS, RPS)], zrow_v)
  pltpu.sync_copy(zrow_v, out_hbm.at[c, pl.ds(s * RPS, RPS)])


def _hist_call(dst2):
  k = pl.kernel(
      _hist_body,
      out_type=jax.ShapeDtypeStruct((NC, NPAD), _f32),
      mesh=_mesh(),
      scratch_types=[
          pltpu.VMEM_SHARED((NPAD,), _f32),
          pltpu.VMEM((RMAX, ECH), _i32),
          pltpu.VMEM((ECH,), _f32),
          pltpu.VMEM((RPS,), _f32),
          pltpu.SemaphoreType.DMA,
      ],
  )
  return k(dst2)


def _seg_body(g_hbm, src_hbm, dst_hbm, out_hbm, s_sh, isrc_v, idst_v,
              rows_a, rows_b, isem, ga_sem, gb_sem, sa_sem, sb_sem):
  c = lax.axis_index("c")
  s = lax.axis_index("s")
  wid = c * NS + s
  nrows, rbase = _tile_rows(wid)

  pltpu.async_copy(src_hbm.at[pl.ds(rbase, PH)], isrc_v, isem)
  pltpu.async_copy(dst_hbm.at[pl.ds(rbase, PH)], idst_v, isem)

  zeros = jnp.zeros((16,), _f32)

  def zfill(t, _):
    rows_a[t // 8, pl.ds((t % 8) * 16, 16)] = zeros
    return 0
  lax.fori_loop(0, 32 * 8, zfill, 0)
  z32 = rows_a.at[pl.ds(0, 32)]
  for p in range(RPS // 32):
    pltpu.async_copy(z32, s_sh.at[pl.ds(s * RPS + p * 32, 32)], isem)
  for p in range(RPS // 32):
    pltpu.make_async_copy(z32, s_sh.at[pl.ds(s * RPS, 32)], isem).wait()
  pltpu.make_async_copy(src_hbm.at[pl.ds(rbase, PH)], isrc_v, isem).wait()
  pltpu.make_async_copy(dst_hbm.at[pl.ds(rbase, PH)], idst_v, isem).wait()
  plsc.subcore_barrier()

  def _gather(i, buf, sem):
    pltpu.async_copy(g_hbm.at[isrc_v.at[i]], buf, sem)

  def _scatter(i, buf, sem):
    pltpu.async_copy(buf, s_sh.at[idst_v.at[i]], sem, add=True)

  def _wait(buf, sem):
    pltpu.make_async_copy(g_hbm.at[isrc_v.at[0]], buf, sem).wait()

  def _run_phase(npairs):
    _gather(0, rows_a, ga_sem)

    def pair(j, _):
      i0 = 2 * j
      i1 = i0 + 1
      _gather(i1, rows_b, gb_sem)
      _wait(rows_a, ga_sem)
      pltpu.sync_copy(rows_a, s_sh.at[idst_v.at[i0]], add=True)

      @pl.when(j < npairs - 1)
      def _():
        _gather(i0 + 2, rows_a, ga_sem)
      _wait(rows_b, gb_sem)
      pltpu.sync_copy(rows_b, s_sh.at[idst_v.at[i1]], add=True)
      return 0
    lax.fori_loop(0, npairs, pair, 0)

  _run_phase(PH // 2)
  pltpu.sync_copy(src_hbm.at[pl.ds(rbase + PH, PH)], isrc_v)
  pltpu.sync_copy(dst_hbm.at[pl.ds(rbase + PH, PH)], idst_v)
  _run_phase((nrows - PH) // 2)

  plsc.subcore_barrier()
  rb = s * RPS
  obufs = ((rows_a, sa_sem), (rows_b, sb_sem))
  npieces = RPS // ECH
  for p in range(npieces):
    buf, ssem = obufs[p % 2]
    if p >= 2:
      pltpu.make_async_copy(buf, out_hbm.at[c, pl.ds(rb, ECH)], ssem).wait()
    pltpu.sync_copy(s_sh.at[pl.ds(rb + p * ECH, ECH)], buf)
    pltpu.async_copy(buf, out_hbm.at[c, pl.ds(rb + p * ECH, ECH)], ssem)
  for p in (npieces - 2, npieces - 1):
    buf, ssem = obufs[p % 2]
    pltpu.make_async_copy(buf, out_hbm.at[c, pl.ds(rb, ECH)], ssem).wait()


def _seg_call(g, src2, dst2):
  k = pl.kernel(
      _seg_body,
      out_type=jax.ShapeDtypeStruct((NC, NPAD, H), _f32),
      mesh=_mesh(),
      scratch_types=[
          pltpu.VMEM_SHARED((NPAD, H), _f32),
          pltpu.VMEM((PH, ECH), _i32),
          pltpu.VMEM((PH, ECH), _i32),
          pltpu.VMEM((ECH, H), _f32),
          pltpu.VMEM((ECH, H), _f32),
          pltpu.SemaphoreType.DMA,
          pltpu.SemaphoreType.DMA,
          pltpu.SemaphoreType.DMA,
          pltpu.SemaphoreType.DMA,
          pltpu.SemaphoreType.DMA,
      ],
  )
  return k(g, src2, dst2)


def _link_body(u_hbm, v_hbm, ps_hbm, pd_hbm, ns_hbm, nd_hbm,
               lp_hbm, ln_hbm, u_v, v_v, is0_v, id0_v, is1_v, id1_v,
               ob_v, ob2_v, lsem):
  c = lax.axis_index("c")
  s = lax.axis_index("s")
  wid = c * NS + s
  base = wid * LCH

  pltpu.async_copy(u_hbm, u_v, lsem)
  pltpu.async_copy(v_hbm, v_v, lsem)
  pltpu.async_copy(ps_hbm.at[pl.ds(base, LCH)], is0_v, lsem)
  pltpu.async_copy(pd_hbm.at[pl.ds(base, LCH)], id0_v, lsem)
  pltpu.async_copy(ns_hbm.at[pl.ds(base, LCH)], is1_v, lsem)
  pltpu.async_copy(nd_hbm.at[pl.ds(base, LCH)], id1_v, lsem)
  for _ in range(2):
    pltpu.make_async_copy(u_hbm, u_v, lsem).wait()
  for _ in range(4):
    pltpu.make_async_copy(ps_hbm.at[pl.ds(base, LCH)], is0_v, lsem).wait()

  for iss, ids, oh, ob in ((is0_v, id0_v, lp_hbm, ob_v),
                           (is1_v, id1_v, ln_hbm, ob2_v)):
    def gath(j, _, iss=iss, ids=ids, ob=ob):
      for t in range(4):
        o = (4 * j + t) * 16
        a = plsc.load_gather(u_v, [iss[pl.ds(o, 16)]])
        b = plsc.load_gather(v_v, [ids[pl.ds(o, 16)]])
        ob[pl.ds(o, 16)] = a + b
      return 0
    lax.fori_loop(0, LCH // 64, gath, 0)
    pltpu.async_copy(ob, oh.at[pl.ds(base, LCH)], lsem)
  for _ in range(2):
    pltpu.make_async_copy(ob_v, lp_hbm.at[pl.ds(base, LCH)], lsem).wait()


def _link_call(u, v, ps, pd, ns_, nd):
  k = pl.kernel(
      _link_body,
      out_type=[jax.ShapeDtypeStruct((EPPAD,), _f32),
                jax.ShapeDtypeStruct((EPPAD,), _f32)],
      mesh=_mesh(),
      scratch_types=[
          pltpu.VMEM((NPAD,), _f32),
          pltpu.VMEM((NPAD,), _f32),
          pltpu.VMEM((LCH,), _i32),
          pltpu.VMEM((LCH,), _i32),
          pltpu.VMEM((LCH,), _i32),
          pltpu.VMEM((LCH,), _i32),
          pltpu.VMEM((LCH,), _f32),
          pltpu.VMEM((LCH,), _f32),
          pltpu.SemaphoreType.DMA,
      ],
      compiler_params=pltpu.CompilerParams(needs_layout_passes=False),
  )
  return k(u, v, ps, pd, ns_, nd)


_BLK = 1024
_GRID = NPAD // _BLK


def _dinv_of(hist_ref):
  return lax.rsqrt(1.0 + hist_ref[0, :] + hist_ref[1, :])[:, None]


def _tc1_body(x_ref, w1_ref, hist_ref, g1_ref):
  h = jnp.dot(x_ref[...], w1_ref[...], preferred_element_type=_f32)
  g1_ref[...] = h * _dinv_of(hist_ref)


def _tc1_call(x_pad, w1, hist):
  return pl.pallas_call(
      _tc1_body,
      grid=(_GRID,),
      in_specs=[
          pl.BlockSpec((_BLK, D), lambda i: (i, 0)),
          pl.BlockSpec((D, H), lambda i: (0, 0)),
          pl.BlockSpec((NC, _BLK), lambda i: (0, i)),
      ],
      out_specs=pl.BlockSpec((_BLK, H), lambda i: (i, 0)),
      out_shape=jax.ShapeDtypeStruct((NPAD, H), _f32),
  )(x_pad, w1, hist)


def _tc2_body(s_ref, g1_ref, hist_ref, b1_ref, w2_ref, g2_ref):
  dinv = _dinv_of(hist_ref)
  o1 = jnp.maximum(
      dinv * (s_ref[0] + s_ref[1] + g1_ref[...]) + b1_ref[...], 0.0)
  h2 = jnp.dot(o1, w2_ref[...], preferred_element_type=_f32)
  g2_ref[...] = h2 * dinv


def _tc2_call(s1, g1, hist, b1, w2):
  return pl.pallas_call(
      _tc2_body,
      grid=(_GRID,),
      in_specs=[
          pl.BlockSpec((NC, _BLK, H), lambda i: (0, i, 0)),
          pl.BlockSpec((_BLK, H), lambda i: (i, 0)),
          pl.BlockSpec((NC, _BLK), lambda i: (0, i)),
          pl.BlockSpec((1, H), lambda i: (0, 0)),
          pl.BlockSpec((H, H), lambda i: (0, 0)),
      ],
      out_specs=pl.BlockSpec((_BLK, H), lambda i: (i, 0)),
      out_shape=jax.ShapeDtypeStruct((NPAD, H), _f32),
  )(s1, g1, hist, b1, w2)


def _tc3_body(s_ref, g2_ref, hist_ref, b2_ref, wc_ref, bc_ref,
              wl_ref, bl_ref, np_ref, u_ref, v_ref):
  dinv = _dinv_of(hist_ref)
  h = dinv * (s_ref[0] + s_ref[1] + g2_ref[...]) + b2_ref[...]
  nh = jnp.maximum(h, 0.0)
  logits = jnp.dot(nh, wc_ref[...], preferred_element_type=_f32) + bc_ref[...]
  m = jnp.max(logits, axis=1, keepdims=True)
  lse = jnp.log(jnp.sum(jnp.exp(logits - m), axis=1, keepdims=True)) + m
  np_ref[...] = logits - lse
  uv = jnp.dot(h, wl_ref[...], preferred_element_type=_f32)
  u_ref[...] = uv[:, 0:1] + bl_ref[...]
  v_ref[...] = uv[:, 1:2]


def _tc3_call(s2, g2, hist, b2, wc, bc, wl2, bl):
  return pl.pallas_call(
      _tc3_body,
      grid=(_GRID,),
      in_specs=[
          pl.BlockSpec((NC, _BLK, H), lambda i: (0, i, 0)),
          pl.BlockSpec((_BLK, H), lambda i: (i, 0)),
          pl.BlockSpec((NC, _BLK), lambda i: (0, i)),
          pl.BlockSpec((1, H), lambda i: (0, 0)),
          pl.BlockSpec((H, C), lambda i: (0, 0)),
          pl.BlockSpec((1, C), lambda i: (0, 0)),
          pl.BlockSpec((H, 2), lambda i: (0, 0)),
          pl.BlockSpec((1, 1), lambda i: (0, 0)),
      ],
      out_specs=[
          pl.BlockSpec((_BLK, C), lambda i: (i, 0)),
          pl.BlockSpec((_BLK, 1), lambda i: (i, 0)),
          pl.BlockSpec((_BLK, 1), lambda i: (i, 0)),
      ],
      out_shape=[
          jax.ShapeDtypeStruct((NPAD, C), _f32),
          jax.ShapeDtypeStruct((NPAD, 1), _f32),
          jax.ShapeDtypeStruct((NPAD, 1), _f32),
      ],
  )(s2, g2, hist, b2, wc, bc, wl2, bl)


def kernel(x, edge_index, edge_index_pos, edge_index_neg,
           W1, b1, W2, b2, Wc, bc, Wl, bl):
  src2 = jnp.pad(edge_index[0].reshape(EROWS, ECH),
                 ((0, ERPAD - EROWS), (0, 0)))
  dst2 = jnp.pad(edge_index[1].reshape(EROWS, ECH),
                 ((0, ERPAD - EROWS), (0, 0)))
  x_pad = jnp.pad(x, ((0, NPAD - N), (0, 0)))

  hist = _hist_call(dst2)

  g1 = _tc1_call(x_pad, W1, hist)
  s1 = _seg_call(g1, src2, dst2)
  g2 = _tc2_call(s1, g1, hist, b1.reshape(1, H), W2)
  s2 = _seg_call(g2, src2, dst2)

  wl2 = jnp.concatenate([Wl[:H], Wl[H:]], axis=1)
  node_pad, u2, v2 = _tc3_call(s2, g2, hist, b2.reshape(1, H), Wc,
                               bc.reshape(1, C), wl2, bl.reshape(1, 1))

  pad = (0, EPPAD - EP)
  lp_pad, ln_pad = _link_call(
      u2[:, 0], v2[:, 0],
      jnp.pad(edge_index_pos[0], pad), jnp.pad(edge_index_pos[1], pad),
      jnp.pad(edge_index_neg[0], pad), jnp.pad(edge_index_neg[1], pad))

  return (node_pad[:N], lp_pad[:EP], ln_pad[:EP])


# speedup vs baseline: 1.2836x; 1.0026x over previous
import functools

import jax
import jax.numpy as jnp
from jax import lax
from jax.experimental import pallas as pl
from jax.experimental.pallas import tpu as pltpu
from jax.experimental.pallas import tpu_sc as plsc

N = 10000
NPAD = 10240
E = 320000
EP = 100000
D = 128
H = 128
C = 10

NC = 2
NS = 16
NW = NC * NS
ECH = 128
EROWS = E // ECH
RMAX = 80
PH = 48
ERPAD = 2560
RPS = NPAD // NS
EPPAD = 100352
LCH = EPPAD // NW

_f32 = jnp.float32
_i32 = jnp.int32


def _mesh():
  return plsc.VectorSubcoreMesh(core_axis_name="c", subcore_axis_name="s")


def _tile_rows(wid):
  nrows = jnp.where(wid < 24, 80, jnp.where(wid == 31, 76, 72))
  rbase = jnp.where(wid < 24, wid * 80, 1920 + (wid - 24) * 72)
  return nrows, rbase


def _hist_body(dst_hbm, out_hbm, hist_sh, idx_v, val_v, zrow_v, isem):
  c = lax.axis_index("c")
  s = lax.axis_index("s")
  wid = c * NS + s
  nrows, rbase = _tile_rows(wid)

  icp = pltpu.async_copy(dst_hbm.at[pl.ds(rbase, RMAX)], idx_v, isem)

  ones = jnp.ones((16,), _f32)
  zeros = jnp.zeros((16,), _f32)
  for j in range(ECH // 16):
    val_v[pl.ds(j * 16, 16)] = ones

  def zfill(j, _):
    zrow_v[pl.ds(j * 16, 16)] = zeros
    return 0
  lax.fori_loop(0, RPS // 16, zfill, 0)

  pltpu.sync_copy(zrow_v, hist_sh.at[pl.ds(s * RPS, RPS)])
  icp.wait()
  plsc.subcore_barrier()

  def _drain():
    pltpu.make_async_copy(val_v, hist_sh.at[idx_v.at[0]], isem).wait()

  def chunk(i, _):
    pltpu.async_copy(val_v, hist_sh.at[idx_v.at[i]], isem, add=True)

    @pl.when(i >= 8)
    def _():
      _drain()
    return 0
  lax.fori_loop(0, nrows, chunk, 0)

  def tail(_k, _):
    _drain()
    return 0
  lax.fori_loop(0, jnp.minimum(nrows, 8), tail, 0)

  plsc.subcore_barrier()
  pltpu.sync_copy(hist_sh.at[pl.ds(s * RPS, RPS)], zrow_v)
  pltpu.sync_copy(zrow_v, out_hbm.at[c, pl.ds(s * RPS, RPS)])


def _hist_call(dst2):
  k = pl.kernel(
      _hist_body,
      out_type=jax.ShapeDtypeStruct((NC, NPAD), _f32),
      mesh=_mesh(),
      scratch_types=[
          pltpu.VMEM_SHARED((NPAD,), _f32),
          pltpu.VMEM((RMAX, ECH), _i32),
          pltpu.VMEM((ECH,), _f32),
          pltpu.VMEM((RPS,), _f32),
          pltpu.SemaphoreType.DMA,
      ],
  )
  return k(dst2)


def _seg_body(g_hbm, src_hbm, dst_hbm, out_hbm, s_sh, isrc_v, idst_v,
              rows_a, rows_b, isem, ga_sem, gb_sem, sa_sem, sb_sem):
  c = lax.axis_index("c")
  s = lax.axis_index("s")
  wid = c * NS + s
  nrows, rbase = _tile_rows(wid)

  pltpu.async_copy(src_hbm.at[pl.ds(rbase, PH)], isrc_v, isem)
  pltpu.async_copy(dst_hbm.at[pl.ds(rbase, PH)], idst_v, isem)

  zeros = jnp.zeros((16,), _f32)

  def zfill(t, _):
    rows_a[t // 8, pl.ds((t % 8) * 16, 16)] = zeros
    return 0
  lax.fori_loop(0, 32 * 8, zfill, 0)
  z32 = rows_a.at[pl.ds(0, 32)]
  for p in range(RPS // 32):
    pltpu.async_copy(z32, s_sh.at[pl.ds(s * RPS + p * 32, 32)], isem)
  for p in range(RPS // 32):
    pltpu.make_async_copy(z32, s_sh.at[pl.ds(s * RPS, 32)], isem).wait()
  pltpu.make_async_copy(src_hbm.at[pl.ds(rbase, PH)], isrc_v, isem).wait()
  pltpu.make_async_copy(dst_hbm.at[pl.ds(rbase, PH)], idst_v, isem).wait()
  plsc.subcore_barrier()

  def _gather(i, buf, sem):
    pltpu.async_copy(g_hbm.at[isrc_v.at[i]], buf, sem)

  def _scatter(i, buf, sem):
    pltpu.async_copy(buf, s_sh.at[idst_v.at[i]], sem, add=True)

  def _wait(buf, sem):
    pltpu.make_async_copy(g_hbm.at[isrc_v.at[0]], buf, sem).wait()

  def _run_phase(npairs):
    _gather(0, rows_a, ga_sem)

    def pair(j, _):
      i0 = 2 * j
      i1 = i0 + 1
      _gather(i1, rows_b, gb_sem)
      _wait(rows_a, ga_sem)
      pltpu.sync_copy(rows_a, s_sh.at[idst_v.at[i0]], add=True)

      @pl.when(j < npairs - 1)
      def _():
        _gather(i0 + 2, rows_a, ga_sem)
      _wait(rows_b, gb_sem)
      pltpu.sync_copy(rows_b, s_sh.at[idst_v.at[i1]], add=True)
      return 0
    lax.fori_loop(0, npairs, pair, 0)

  _run_phase(PH // 2)
  pltpu.sync_copy(src_hbm.at[pl.ds(rbase + PH, PH)], isrc_v)
  pltpu.sync_copy(dst_hbm.at[pl.ds(rbase + PH, PH)], idst_v)
  _run_phase((nrows - PH) // 2)

  plsc.subcore_barrier()
  rb = s * RPS
  obufs = ((rows_a, sa_sem), (rows_b, sb_sem))
  npieces = RPS // ECH
  for p in range(npieces):
    buf, ssem = obufs[p % 2]
    if p >= 2:
      pltpu.make_async_copy(buf, out_hbm.at[c, pl.ds(rb, ECH)], ssem).wait()
    pltpu.sync_copy(s_sh.at[pl.ds(rb + p * ECH, ECH)], buf)
    pltpu.async_copy(buf, out_hbm.at[c, pl.ds(rb + p * ECH, ECH)], ssem)
  for p in (npieces - 2, npieces - 1):
    buf, ssem = obufs[p % 2]
    pltpu.make_async_copy(buf, out_hbm.at[c, pl.ds(rb, ECH)], ssem).wait()


def _seg_call(g, src2, dst2):
  k = pl.kernel(
      _seg_body,
      out_type=jax.ShapeDtypeStruct((NC, NPAD, H), _f32),
      mesh=_mesh(),
      scratch_types=[
          pltpu.VMEM_SHARED((NPAD, H), _f32),
          pltpu.VMEM((PH, ECH), _i32),
          pltpu.VMEM((PH, ECH), _i32),
          pltpu.VMEM((ECH, H), _f32),
          pltpu.VMEM((ECH, H), _f32),
          pltpu.SemaphoreType.DMA,
          pltpu.SemaphoreType.DMA,
          pltpu.SemaphoreType.DMA,
          pltpu.SemaphoreType.DMA,
          pltpu.SemaphoreType.DMA,
      ],
  )
  return k(g, src2, dst2)


def _link_body(u_hbm, v_hbm, ps_hbm, pd_hbm, ns_hbm, nd_hbm,
               lp_hbm, ln_hbm, u_v, v_v, is0_v, id0_v, is1_v, id1_v,
               ob_v, ob2_v, lsem):
  c = lax.axis_index("c")
  s = lax.axis_index("s")
  wid = c * NS + s
  base = wid * LCH

  pltpu.async_copy(u_hbm, u_v, lsem)
  pltpu.async_copy(v_hbm, v_v, lsem)
  pltpu.async_copy(ps_hbm.at[pl.ds(base, LCH)], is0_v, lsem)
  pltpu.async_copy(pd_hbm.at[pl.ds(base, LCH)], id0_v, lsem)
  pltpu.async_copy(ns_hbm.at[pl.ds(base, LCH)], is1_v, lsem)
  pltpu.async_copy(nd_hbm.at[pl.ds(base, LCH)], id1_v, lsem)
  for _ in range(2):
    pltpu.make_async_copy(u_hbm, u_v, lsem).wait()
  for _ in range(4):
    pltpu.make_async_copy(ps_hbm.at[pl.ds(base, LCH)], is0_v, lsem).wait()

  for iss, ids, oh, ob in ((is0_v, id0_v, lp_hbm, ob_v),
                           (is1_v, id1_v, ln_hbm, ob2_v)):
    def gath(j, _, iss=iss, ids=ids, ob=ob):
      for t in range(4):
        o = (4 * j + t) * 16
        a = plsc.load_gather(u_v, [iss[pl.ds(o, 16)]])
        b = plsc.load_gather(v_v, [ids[pl.ds(o, 16)]])
        ob[pl.ds(o, 16)] = a + b
      return 0
    lax.fori_loop(0, LCH // 64, gath, 0)
    pltpu.async_copy(ob, oh.at[pl.ds(base, LCH)], lsem)
  for _ in range(2):
    pltpu.make_async_copy(ob_v, lp_hbm.at[pl.ds(base, LCH)], lsem).wait()


def _link_call(u, v, ps, pd, ns_, nd):
  k = pl.kernel(
      _link_body,
      out_type=[jax.ShapeDtypeStruct((EPPAD,), _f32),
                jax.ShapeDtypeStruct((EPPAD,), _f32)],
      mesh=_mesh(),
      scratch_types=[
          pltpu.VMEM((NPAD,), _f32),
          pltpu.VMEM((NPAD,), _f32),
          pltpu.VMEM((LCH,), _i32),
          pltpu.VMEM((LCH,), _i32),
          pltpu.VMEM((LCH,), _i32),
          pltpu.VMEM((LCH,), _i32),
          pltpu.VMEM((LCH,), _f32),
          pltpu.VMEM((LCH,), _f32),
          pltpu.SemaphoreType.DMA,
      ],
      compiler_params=pltpu.CompilerParams(needs_layout_passes=False),
  )
  return k(u, v, ps, pd, ns_, nd)


_BLK = 1024
_GRID = NPAD // _BLK


def _dinv_of(hist_ref):
  return lax.rsqrt(1.0 + hist_ref[0, :] + hist_ref[1, :])[:, None]


def _tc1_body(x_ref, w1_ref, hist_ref, g1_ref):
  h = jnp.dot(x_ref[...], w1_ref[...], preferred_element_type=_f32)
  g1_ref[...] = h * _dinv_of(hist_ref)


def _tc1_call(x_pad, w1, hist):
  return pl.pallas_call(
      _tc1_body,
      grid=(_GRID,),
      in_specs=[
          pl.BlockSpec((_BLK, D), lambda i: (i, 0)),
          pl.BlockSpec((D, H), lambda i: (0, 0)),
          pl.BlockSpec((NC, _BLK), lambda i: (0, i)),
      ],
      out_specs=pl.BlockSpec((_BLK, H), lambda i: (i, 0)),
      out_shape=jax.ShapeDtypeStruct((NPAD, H), _f32),
  )(x_pad, w1, hist)


def _tc2_body(s_ref, g1_ref, hist_ref, b1_ref, w2_ref, g2_ref):
  dinv = _dinv_of(hist_ref)
  o1 = jnp.maximum(
      dinv * (s_ref[0] + s_ref[1] + g1_ref[...]) + b1_ref[...], 0.0)
  h2 = jnp.dot(o1, w2_ref[...], preferred_element_type=_f32)
  g2_ref[...] = h2 * dinv


def _tc2_call(s1, g1, hist, b1, w2):
  return pl.pallas_call(
      _tc2_body,
      grid=(_GRID,),
      in_specs=[
          pl.BlockSpec((NC, _BLK, H), lambda i: (0, i, 0)),
          pl.BlockSpec((_BLK, H), lambda i: (i, 0)),
          pl.BlockSpec((NC, _BLK), lambda i: (0, i)),
          pl.BlockSpec((1, H), lambda i: (0, 0)),
          pl.BlockSpec((H, H), lambda i: (0, 0)),
      ],
      out_specs=pl.BlockSpec((_BLK, H), lambda i: (i, 0)),
      out_shape=jax.ShapeDtypeStruct((NPAD, H), _f32),
  )(s1, g1, hist, b1, w2)


def _tc3_body(s_ref, g2_ref, hist_ref, b2_ref, wc_ref, bc_ref,
              wl_ref, bl_ref, np_ref, u_ref, v_ref):
  dinv = _dinv_of(hist_ref)
  h = dinv * (s_ref[0] + s_ref[1] + g2_ref[...]) + b2_ref[...]
  nh = jnp.maximum(h, 0.0)
  logits = jnp.dot(nh, wc_ref[...], preferred_element_type=_f32) + bc_ref[...]
  m = jnp.max(logits, axis=1, keepdims=True)
  lse = jnp.log(jnp.sum(jnp.exp(logits - m), axis=1, keepdims=True)) + m
  np_ref[...] = logits - lse
  uv = jnp.dot(h, wl_ref[...], preferred_element_type=_f32)
  u_ref[...] = uv[:, 0:1] + bl_ref[...]
  v_ref[...] = uv[:, 1:2]


def _tc3_call(s2, g2, hist, b2, wc, bc, wl2, bl):
  return pl.pallas_call(
      _tc3_body,
      grid=(_GRID,),
      in_specs=[
          pl.BlockSpec((NC, _BLK, H), lambda i: (0, i, 0)),
          pl.BlockSpec((_BLK, H), lambda i: (i, 0)),
          pl.BlockSpec((NC, _BLK), lambda i: (0, i)),
          pl.BlockSpec((1, H), lambda i: (0, 0)),
          pl.BlockSpec((H, C), lambda i: (0, 0)),
          pl.BlockSpec((1, C), lambda i: (0, 0)),
          pl.BlockSpec((H, 2), lambda i: (0, 0)),
          pl.BlockSpec((1, 1), lambda i: (0, 0)),
      ],
      out_specs=[
          pl.BlockSpec((_BLK, C), lambda i: (i, 0)),
          pl.BlockSpec((_BLK, 1), lambda i: (i, 0)),
          pl.BlockSpec((_BLK, 1), lambda i: (i, 0)),
      ],
      out_shape=[
          jax.ShapeDtypeStruct((NPAD, C), _f32),
          jax.ShapeDtypeStruct((NPAD, 1), _f32),
          jax.ShapeDtypeStruct((NPAD, 1), _f32),
      ],
  )(s2, g2, hist, b2, wc, bc, wl2, bl)


def kernel(x, edge_index, edge_index_pos, edge_index_neg,
           W1, b1, W2, b2, Wc, bc, Wl, bl):
  src2 = jnp.pad(edge_index[0].reshape(EROWS, ECH),
                 ((0, ERPAD - EROWS), (0, 0)))
  dst2 = jnp.pad(edge_index[1].reshape(EROWS, ECH),
                 ((0, ERPAD - EROWS), (0, 0)))
  x_pad = jnp.pad(x, ((0, NPAD - N), (0, 0)))

  hist = _hist_call(dst2)

  g1 = _tc1_call(x_pad, W1, hist)
  s1 = _seg_call(g1, src2, dst2)
  g2 = _tc2_call(s1, g1, hist, b1.reshape(1, H), W2)
  s2 = _seg_call(g2, src2, dst2)

  wl2 = jnp.concatenate([Wl[:H], Wl[H:]], axis=1)
  node_pad, u2, v2 = _tc3_call(s2, g2, hist, b2.reshape(1, H), Wc,
                               bc.reshape(1, C), wl2, bl.reshape(1, 1))

  pad = (0, EPPAD - EP)
  lp_pad, ln_pad = _link_call(
      u2[:, 0], v2[:, 0],
      jnp.pad(edge_index_pos[0], pad), jnp.pad(edge_index_pos[1], pad),
      jnp.pad(edge_index_neg[0], pad), jnp.pad(edge_index_neg[1], pad))

  return (node_pad[:N], lp_pad[:EP], ln_pad[:EP])
